# baseline (speedup 1.0000x reference)
import functools

import numpy as np
import jax
import jax.numpy as jnp
from jax.experimental import pallas as pl
from jax.experimental.pallas import tpu as pltpu

_CQ = 128


@functools.lru_cache(maxsize=None)
def _spread_matrix(gg: int, cq: int):
    g = np.zeros((cq, gg * cq), dtype=np.float32)
    for m in range(cq):
        g[m, gg * m] = 1.0
    return g


def _spp_kernel(x_ref, g4_ref, g2_ref, o1_ref, o2_ref, o4_ref, *, h, w, c):
    t_id = pl.program_id(0)
    bh = h // 4
    bw = w // 4
    inv_hw = 1.0 / float(h * w)
    inv2 = 4.0 * inv_hw
    inv4 = 16.0 * inv_hw
    nq = c // _CQ

    s4 = []
    for j in range(4):
        acc = None
        for a in range(bh):
            for b in range(bw * j, bw * (j + 1)):
                v = x_ref[a * w + b]
                acc = v if acc is None else acc + v
        s4.append(acc)

    o1_ref[0] = (s4[0] + s4[1] + s4[2] + s4[3]) * inv_hw

    out2 = None
    for j2 in range(2):
        p2 = (s4[2 * j2] + s4[2 * j2 + 1]) * inv2
        chunks = []
        for q in range(nq):
            chunks.append(jnp.dot(p2[:, _CQ * q:_CQ * (q + 1)], g2_ref[...],
                                  preferred_element_type=jnp.float32))
        v = jnp.concatenate(chunks, axis=1)
        v = pltpu.roll(v, j2, 1) if j2 else v
        out2 = v if out2 is None else out2 + v
    o2_ref[0] = pltpu.roll(out2, 2 * (t_id // 2), 1)

    out4 = None
    for j in range(4):
        p4 = s4[j] * inv4
        chunks = []
        for q in range(nq):
            chunks.append(jnp.dot(p4[:, _CQ * q:_CQ * (q + 1)], g4_ref[...],
                                  preferred_element_type=jnp.float32))
        v = jnp.concatenate(chunks, axis=1)
        v = pltpu.roll(v, j, 1) if j else v
        out4 = v if out4 is None else out4 + v
    o4_ref[0] = pltpu.roll(out4, 4 * t_id, 1)


def _assemble_kernel(o1_ref, o2_ref, o4_ref, out_ref, *, c):
    v1 = o1_ref[0] + o1_ref[1] + o1_ref[2] + o1_ref[3]
    v2 = o2_ref[0] + o2_ref[1] + o2_ref[2] + o2_ref[3]
    v4 = o4_ref[0] + o4_ref[1] + o4_ref[2] + o4_ref[3]
    rows = 21 * c // 128
    r1 = c // 128
    r2 = 5 * c // 128
    for j in range(r1):
        out_ref[j::rows, :] = v1[:, 128 * j:128 * (j + 1)]
    for j in range(r2 - r1):
        out_ref[r1 + j::rows, :] = v2[:, 128 * j:128 * (j + 1)]
    for j in range(rows - r2):
        out_ref[r2 + j::rows, :] = v4[:, 128 * j:128 * (j + 1)]


def kernel(x):
    B, C, H, W = x.shape
    HW = H * W
    xt = jnp.transpose(x, (2, 3, 0, 1)).reshape(HW, B, C)
    dt = x.dtype

    g4_op = jnp.asarray(_spread_matrix(16, _CQ))
    g2_op = jnp.asarray(_spread_matrix(4, _CQ))

    o1, o2, o4 = pl.pallas_call(
        functools.partial(_spp_kernel, h=H, w=W, c=C),
        out_shape=[
            jax.ShapeDtypeStruct((4, B, C), dt),
            jax.ShapeDtypeStruct((4, B, 4 * C), dt),
            jax.ShapeDtypeStruct((4, B, 16 * C), dt),
        ],
        grid=(4,),
        in_specs=[
            pl.BlockSpec((HW // 4, B, C), lambda t: (t, 0, 0)),
            pl.BlockSpec((_CQ, 16 * _CQ), lambda t: (0, 0)),
            pl.BlockSpec((_CQ, 4 * _CQ), lambda t: (0, 0)),
        ],
        out_specs=[
            pl.BlockSpec((1, B, C), lambda t: (t, 0, 0)),
            pl.BlockSpec((1, B, 4 * C), lambda t: (t, 0, 0)),
            pl.BlockSpec((1, B, 16 * C), lambda t: (t, 0, 0)),
        ],
        compiler_params=pltpu.CompilerParams(
            dimension_semantics=("parallel",),
            vmem_limit_bytes=64 * 1024 * 1024,
        ),
    )(xt, g4_op, g2_op)

    rows = 21 * C // 128
    bh = B // 2
    out2d = pl.pallas_call(
        functools.partial(_assemble_kernel, c=C),
        out_shape=jax.ShapeDtypeStruct((B * rows, 128), dt),
        grid=(2,),
        in_specs=[
            pl.BlockSpec((4, bh, C), lambda i: (0, i, 0)),
            pl.BlockSpec((4, bh, 4 * C), lambda i: (0, i, 0)),
            pl.BlockSpec((4, bh, 16 * C), lambda i: (0, i, 0)),
        ],
        out_specs=pl.BlockSpec((bh * rows, 128), lambda i: (i, 0)),
        compiler_params=pltpu.CompilerParams(
            dimension_semantics=("parallel",),
            vmem_limit_bytes=64 * 1024 * 1024,
        ),
    )(o1, o2, o4)
    return out2d.reshape(B, 21 * C, 1, 1)


# speedup vs baseline: 1.1276x; 1.1276x over previous
import functools

import numpy as np
import jax
import jax.numpy as jnp
from jax.experimental import pallas as pl
from jax.experimental.pallas import tpu as pltpu

_CQ = 128
_BB = 16


def _window_starts(n: int, g: int):
    return [((i * n) // g, -((-(i + 1) * n) // g)) for i in range(g)]


@functools.lru_cache(maxsize=None)
def _spread_matrix(gg: int, grp: int, cq: int):
    g = np.zeros((grp * cq, gg * cq), dtype=np.float32)
    for k in range(grp):
        for m in range(cq):
            g[cq * k + m, gg * m + k] = 1.0
    return g


def _spp_kernel(x_ref, g4_ref, g2_ref, o1_ref, o2_ref, o4_ref, *, h, w):
    hs4 = _window_starts(h, 4)
    ws4 = _window_starts(w, 4)
    inv_hw = 1.0 / float(h * w)
    inv2 = 4.0 * inv_hw
    inv4 = 16.0 * inv_hw

    s4 = []
    for i in range(4):
        for j in range(4):
            acc = None
            for a in range(hs4[i][0], hs4[i][1]):
                for b in range(ws4[j][0], ws4[j][1]):
                    t = x_ref[a * w + b]
                    acc = t if acc is None else acc + t
            s4.append(acc)
    z2 = [s4[4 * (2 * i2) + 2 * j2] + s4[4 * (2 * i2) + 2 * j2 + 1]
          + s4[4 * (2 * i2 + 1) + 2 * j2] + s4[4 * (2 * i2 + 1) + 2 * j2 + 1]
          for i2 in range(2) for j2 in range(2)]
    z1 = z2[0] + z2[1] + z2[2] + z2[3]

    o1_ref[...] = z1 * inv_hw

    out2 = None
    for k in range(4):
        v = jnp.dot(z2[k], g2_ref[...], preferred_element_type=jnp.float32)
        v = pltpu.roll(v, k, 1) if k else v
        out2 = v if out2 is None else out2 + v
    o2_ref[...] = out2 * inv2

    out4 = None
    for k in range(16):
        v = jnp.dot(s4[k], g4_ref[...], preferred_element_type=jnp.float32)
        v = pltpu.roll(v, k, 1) if k else v
        out4 = v if out4 is None else out4 + v
    o4_ref[...] = out4 * inv4


def _assemble_kernel(o1_ref, o2_ref, o4_ref, out_ref, *, c):
    rows = 21 * c // 128
    r1 = c // 128
    r2 = 5 * c // 128
    for j in range(r1):
        out_ref[j::rows, :] = o1_ref[:, 128 * j:128 * (j + 1)]
    for j in range(r2 - r1):
        out_ref[r1 + j::rows, :] = o2_ref[:, 128 * j:128 * (j + 1)]
    for j in range(rows - r2):
        out_ref[r2 + j::rows, :] = o4_ref[:, 128 * j:128 * (j + 1)]


def kernel(x):
    B, C, H, W = x.shape
    HW = H * W
    xt = jnp.transpose(x, (2, 3, 0, 1)).reshape(HW, B, C)

    nq = C // _CQ
    bb = _BB if B % _BB == 0 else B
    nb = B // bb
    dt = x.dtype

    g4_op = jnp.asarray(_spread_matrix(16, 1, _CQ))
    g2_op = jnp.asarray(_spread_matrix(4, 1, _CQ))

    o1, o2, o4 = pl.pallas_call(
        functools.partial(_spp_kernel, h=H, w=W),
        out_shape=[
            jax.ShapeDtypeStruct((B, C), dt),
            jax.ShapeDtypeStruct((B, 4 * C), dt),
            jax.ShapeDtypeStruct((B, 16 * C), dt),
        ],
        grid=(nb, nq),
        in_specs=[
            pl.BlockSpec((HW, bb, _CQ), lambda b, q: (0, b, q)),
            pl.BlockSpec((_CQ, 16 * _CQ), lambda b, q: (0, 0)),
            pl.BlockSpec((_CQ, 4 * _CQ), lambda b, q: (0, 0)),
        ],
        out_specs=[
            pl.BlockSpec((bb, _CQ), lambda b, q: (b, q)),
            pl.BlockSpec((bb, 4 * _CQ), lambda b, q: (b, q)),
            pl.BlockSpec((bb, 16 * _CQ), lambda b, q: (b, q)),
        ],
        compiler_params=pltpu.CompilerParams(
            dimension_semantics=("parallel", "parallel"),
            vmem_limit_bytes=64 * 1024 * 1024,
        ),
    )(xt, g4_op, g2_op)

    rows = 21 * C // 128
    bh = B // 2
    out2d = pl.pallas_call(
        functools.partial(_assemble_kernel, c=C),
        out_shape=jax.ShapeDtypeStruct((B * rows, 128), dt),
        grid=(2,),
        in_specs=[
            pl.BlockSpec((bh, C), lambda i: (i, 0)),
            pl.BlockSpec((bh, 4 * C), lambda i: (i, 0)),
            pl.BlockSpec((bh, 16 * C), lambda i: (i, 0)),
        ],
        out_specs=pl.BlockSpec((bh * rows, 128), lambda i: (i, 0)),
        compiler_params=pltpu.CompilerParams(
            dimension_semantics=("parallel",),
            vmem_limit_bytes=64 * 1024 * 1024,
        ),
    )(o1, o2, o4)
    return out2d.reshape(B, 21 * C, 1, 1)
